# baseline (speedup 1.0000x reference)
import jax
import jax.numpy as jnp
from jax import lax
from jax.experimental import pallas as pl
from jax.experimental.pallas import tpu as pltpu
from jax.experimental.pallas import tpu_sc as plsc

N_NODES = 10000
N_EDGES = 160000
F = 16
NEG_SLOPE = 0.01

NC, NS = 2, 16
NW = NC * NS
EW = N_EDGES // NW
CH = 128
NFULL = EW // CH
TAIL = EW - NFULL * CH
ROWS2 = 2 * N_NODES
DUMP = ROWS2
SROWS = ROWS2 + 96
ZPT = SROWS // NS
WPT = 1248
PAYW = 2 * F

FPT = 624
FLAST = N_NODES - 15 * FPT

_CHUNKS = [(i * CH, CH) for i in range(NFULL)] + ([(NFULL * CH, TAIL)] if TAIL else [])

_mesh = plsc.VectorSubcoreMesh(
    core_axis_name="c", subcore_axis_name="s", num_cores=NC, num_subcores=NS)


def _gather_body(feat_h, src_h, dst_h, hsrc_h, zdst_h,
                 idxs_v, idxd_v, rs0, rs1, rd0, rd1,
                 gs0, gs1, gd0, gd1, ss0, ss1, sd0, sd1):
  c = lax.axis_index("c")
  s = lax.axis_index("s")
  base = (c * NS + s) * EW

  pltpu.sync_copy(src_h.at[pl.ds(base, EW)], idxs_v)
  pltpu.sync_copy(dst_h.at[pl.ds(base, EW)], idxd_v)

  rs = (rs0, rs1)
  rd = (rd0, rd1)
  gsem = ((gs0, gs1), (gd0, gd1))
  ssem = ((ss0, ss1), (sd0, sd1))
  n = len(_CHUNKS)
  g = [[None, None], [None, None]]
  st = [[None, None], [None, None]]

  def start_gather(i):
    b = i & 1
    off, sz = _CHUNKS[i]
    g[0][b] = pltpu.async_copy(
        feat_h.at[idxs_v.at[pl.ds(off, sz)]], rs[b].at[pl.ds(0, sz)], gsem[0][b])
    g[1][b] = pltpu.async_copy(
        feat_h.at[idxd_v.at[pl.ds(off, sz)]], rd[b].at[pl.ds(0, sz)], gsem[1][b])

  start_gather(0)
  for i in range(n):
    b = i & 1
    g[0][b].wait()
    g[1][b].wait()
    if i + 1 < n:
      if st[0][1 - b] is not None:
        st[0][1 - b].wait()
        st[1][1 - b].wait()
      start_gather(i + 1)
    off, sz = _CHUNKS[i]
    st[0][b] = pltpu.async_copy(
        rs[b].at[pl.ds(0, sz)], hsrc_h.at[pl.ds(base + off, sz)], ssem[0][b])
    st[1][b] = pltpu.async_copy(
        rd[b].at[pl.ds(0, sz)], zdst_h.at[pl.ds(base + off, sz)], ssem[1][b])
  for j in range(2):
    for b in range(2):
      if st[j][b] is not None:
        st[j][b].wait()


def _sc_gather(feat, src, dst):
  return pl.kernel(
      _gather_body,
      out_type=[jax.ShapeDtypeStruct((N_EDGES, F), jnp.float32),
                jax.ShapeDtypeStruct((N_EDGES, F), jnp.float32)],
      mesh=_mesh,
      compiler_params=pltpu.CompilerParams(use_tc_tiling_on_sc=False),
      scratch_types=[
          pltpu.VMEM((EW,), jnp.int32),
          pltpu.VMEM((EW,), jnp.int32),
          pltpu.VMEM((CH, F), jnp.float32),
          pltpu.VMEM((CH, F), jnp.float32),
          pltpu.VMEM((CH, F), jnp.float32),
          pltpu.VMEM((CH, F), jnp.float32),
      ] + [pltpu.SemaphoreType.DMA] * 8,
  )(feat, src, dst)


BK2 = 1000


def _k2_body(h_ref, z_ref, ef_ref, et_ref, was_ref, wad_ref, ba_ref,
             wc0_ref, wc1_ref, bc0_ref, bc1_ref, rm_ref, tm_ref, out_ref):
  h = h_ref[...]
  z = z_ref[...]
  ef = ef_ref[...]
  a = (jnp.dot(h, was_ref[...], preferred_element_type=jnp.float32) +
       jnp.dot(z, wad_ref[...], preferred_element_type=jnp.float32) +
       ba_ref[...])
  a = jnp.where(a >= 0.0, a, NEG_SLOPE * a)
  ex = jnp.exp(a)
  p = (jnp.dot(ef, rm_ref[...], preferred_element_type=jnp.float32) *
       jnp.dot(h, tm_ref[...], preferred_element_type=jnp.float32))
  v0 = (jnp.dot(p, wc0_ref[...], preferred_element_type=jnp.float32) +
        jnp.dot(h, bc0_ref[...], preferred_element_type=jnp.float32))
  v1 = (jnp.dot(p, wc1_ref[...], preferred_element_type=jnp.float32) +
        jnp.dot(h, bc1_ref[...], preferred_element_type=jnp.float32))
  m0 = (et_ref[...] == 0).astype(jnp.float32)
  vsel = m0 * v0 + (1.0 - m0) * v1
  out_ref[:, 0:F] = ex * vsel
  lane = lax.broadcasted_iota(jnp.int32, (BK2, F), 1)
  out_ref[:, F:2 * F] = jnp.where(lane == 0, ex, 0.0)


def _tc_dense(hsrc, zdst, efeat, et2, was, wad, ba, wc0, wc1, bc0, bc1,
              rm, tm):
  grid = (N_EDGES // BK2,)
  return pl.pallas_call(
      _k2_body,
      grid=grid,
      in_specs=[
          pl.BlockSpec((BK2, F), lambda i: (i, 0)),
          pl.BlockSpec((BK2, F), lambda i: (i, 0)),
          pl.BlockSpec((BK2, F), lambda i: (i, 0)),
          pl.BlockSpec((BK2, 1), lambda i: (i, 0)),
          pl.BlockSpec((F, 1), lambda i: (0, 0)),
          pl.BlockSpec((F, 1), lambda i: (0, 0)),
          pl.BlockSpec((1, 1), lambda i: (0, 0)),
          pl.BlockSpec((F * F, F), lambda i: (0, 0)),
          pl.BlockSpec((F * F, F), lambda i: (0, 0)),
          pl.BlockSpec((F, F), lambda i: (0, 0)),
          pl.BlockSpec((F, F), lambda i: (0, 0)),
          pl.BlockSpec((F, F * F), lambda i: (0, 0)),
          pl.BlockSpec((F, F * F), lambda i: (0, 0)),
      ],
      out_specs=pl.BlockSpec((BK2, PAYW), lambda i: (i, 0)),
      out_shape=jax.ShapeDtypeStruct((N_EDGES, PAYW), jnp.float32),
  )(hsrc, zdst, efeat, et2, was, wad, ba, wc0, wc1, bc0, bc1, rm, tm)


def _scatter_body(pay_h, dst_h, et_h, spart_h,
                  idxd_v, idxt_v, r0, r1, rt, p0, p1, sacc,
                  ps0, ps1, cs0, cs1):
  c = lax.axis_index("c")
  s = lax.axis_index("s")
  base = (c * NS + s) * EW
  for i in range(CH):
    p0[i, pl.ds(0, 16)] = jnp.zeros((16,), jnp.float32)
    p0[i, pl.ds(16, 16)] = jnp.zeros((16,), jnp.float32)
  for j in range(9):
    pltpu.sync_copy(p0.at[pl.ds(0, CH)],
                    sacc.at[pl.ds(s * ZPT + j * CH, CH)])
  pltpu.sync_copy(p0.at[pl.ds(0, ZPT - 9 * CH)],
                  sacc.at[pl.ds(s * ZPT + 9 * CH, ZPT - 9 * CH)])

  plsc.subcore_barrier()

  pltpu.sync_copy(dst_h.at[pl.ds(base, EW)], idxd_v.at[pl.ds(0, EW)])
  pltpu.sync_copy(et_h.at[pl.ds(base, EW)], idxt_v.at[pl.ds(0, EW)])

  rbuf = (r0, r1)
  pbuf = (p0, p1)
  psem = (ps0, ps1)
  csem = (cs0, cs1)
  n = len(_CHUNKS)
  pd = [None, None]
  cd = [None, None]

  def compute_ridx(i, rb):
    off = i * CH
    for k in range(CH // 16):
      d16 = idxd_v[pl.ds(off + k * 16, 16)]
      t16 = idxt_v[pl.ds(off + k * 16, 16)]
      rb[pl.ds(k * 16, 16)] = d16 + t16 * N_NODES

  def compute_tail_ridx():
    d16 = idxd_v[pl.ds(NFULL * CH, 16)]
    t16 = idxt_v[pl.ds(NFULL * CH, 16)]
    lane = lax.iota(jnp.int32, 16)
    rt[...] = jnp.where(lane < TAIL, d16 + t16 * N_NODES, DUMP)

  def start_pay(i):
    b = i & 1
    off, sz = _CHUNKS[i]
    pd[b] = pltpu.async_copy(
        pay_h.at[pl.ds(base + off, sz)], pbuf[b].at[pl.ds(0, sz)], psem[b])

  start_pay(0)
  compute_ridx(0, r0)
  for i in range(n):
    b = i & 1
    if i + 1 < n:
      if cd[1 - b] is not None:
        cd[1 - b].wait()
      start_pay(i + 1)
      if TAIL and i + 1 == n - 1:
        compute_tail_ridx()
      else:
        compute_ridx(i + 1, rbuf[1 - b])
    pd[b].wait()
    _, sz = _CHUNKS[i]
    if sz == CH:
      cd[b] = pltpu.async_copy(
          pbuf[b].at[pl.ds(0, CH)], sacc.at[rbuf[b]], csem[b], add=True)
    else:
      cd[b] = pltpu.async_copy(
          pbuf[b].at[pl.ds(0, 16)], sacc.at[rt], csem[b], add=True)
  for b in range(2):
    if cd[b] is not None:
      cd[b].wait()

  plsc.subcore_barrier()

  @pl.when(s < NS - 1)
  def _wb():
    pltpu.sync_copy(sacc.at[pl.ds(s * WPT, WPT)],
                    spart_h.at[c, pl.ds(s * WPT, WPT)])

  @pl.when(s == NS - 1)
  def _wb_last():
    pltpu.sync_copy(sacc.at[pl.ds(15 * WPT, ROWS2 - 15 * WPT)],
                    spart_h.at[c, pl.ds(15 * WPT, ROWS2 - 15 * WPT)])


def _sc_scatter(payload, dst, et):
  return pl.kernel(
      _scatter_body,
      out_type=jax.ShapeDtypeStruct((NC, ROWS2, PAYW), jnp.float32),
      mesh=_mesh,
      compiler_params=pltpu.CompilerParams(use_tc_tiling_on_sc=False),
      scratch_types=[
          pltpu.VMEM((EW + 16 - TAIL,), jnp.int32),
          pltpu.VMEM((EW + 16 - TAIL,), jnp.int32),
          pltpu.VMEM((CH,), jnp.int32),
          pltpu.VMEM((CH,), jnp.int32),
          pltpu.VMEM((16,), jnp.int32),
          pltpu.VMEM((CH, PAYW), jnp.float32),
          pltpu.VMEM((CH, PAYW), jnp.float32),
          pltpu.VMEM_SHARED((SROWS, PAYW), jnp.float32),
      ] + [pltpu.SemaphoreType.DMA] * 4,
  )(payload, dst, et)


def _k4_body(sp0_ref, sp1_ref, bias_ref, out_ref):
  x0 = sp0_ref[0] + sp0_ref[1]
  x1 = sp1_ref[0] + sp1_ref[1]
  m0 = x0[:, 0:F]
  d0 = x0[:, F:F + 1]
  m1 = x1[:, 0:F]
  d1 = x1[:, F:F + 1]
  out_ref[...] = (m0 / jnp.where(d0 > 0, d0, 1.0) +
                  m1 / jnp.where(d1 > 0, d1, 1.0) + bias_ref[...])


def _tc_finalize(spart, bias2):
  return pl.pallas_call(
      _k4_body,
      grid=(1,),
      in_specs=[pl.BlockSpec((NC, N_NODES, PAYW), lambda i: (0, 0, 0)),
                pl.BlockSpec((NC, N_NODES, PAYW), lambda i: (0, 1, 0)),
                pl.BlockSpec((1, F), lambda i: (0, 0))],
      out_specs=pl.BlockSpec((N_NODES, F), lambda i: (0, 0)),
      out_shape=jax.ShapeDtypeStruct((N_NODES, F), jnp.float32),
  )(spart, spart, bias2)


def kernel(feat, efeat, W_attn, b_attn, W_e1, b_e1, W_e2, b_e2, bias,
           edge_index, etype):
  src = edge_index[0].astype(jnp.int32)
  dst = edge_index[1].astype(jnp.int32)
  et = etype.astype(jnp.int32)
  hsrc, zdst = _sc_gather(feat, src, dst)
  wc0 = W_e1.reshape(F * F, F)
  wc1 = W_e2.reshape(F * F, F)
  bc0 = b_e1.reshape(F, F)
  bc1 = b_e2.reshape(F, F)
  was = W_attn[0:F]
  wad = W_attn[F:2 * F]
  ba = b_attn.reshape(1, 1)
  col = jnp.arange(F * F, dtype=jnp.int32)[None, :]
  row = jnp.arange(F, dtype=jnp.int32)[:, None]
  rm = (col // F == row).astype(jnp.float32)
  tm = (col % F == row).astype(jnp.float32)
  payload = _tc_dense(hsrc, zdst, efeat, et.reshape(-1, 1), was, wad, ba,
                      wc0, wc1, bc0, bc1, rm, tm)
  spart = _sc_scatter(payload, dst, et)
  return _tc_finalize(spart, bias.reshape(1, F))


# speedup vs baseline: 1.7014x; 1.7014x over previous
import jax
import jax.numpy as jnp
from jax import lax
from jax.experimental import pallas as pl
from jax.experimental.pallas import tpu as pltpu
from jax.experimental.pallas import tpu_sc as plsc

N_NODES = 10000
N_EDGES = 160000
F = 16
NEG_SLOPE = 0.01

NC, NS = 2, 16
NW = NC * NS
RQ = N_EDGES // 4
WR = 1248
WR_LAST = RQ - (NW - 1) * WR
RCH = 32
NCH = WR // RCH
NCH_LAST = WR_LAST // RCH
IQS = WR_LAST
ROWS2 = 2 * N_NODES
SROWS = ROWS2 + 96
ZPT = SROWS // NS
WPT = 1248
PAYW = 2 * F

_mesh = plsc.VectorSubcoreMesh(
    core_axis_name="c", subcore_axis_name="s", num_cores=NC, num_subcores=NS)
_sc_params = pltpu.CompilerParams(use_tc_tiling_on_sc=False)


def _gather_body(feat_h, src_h, dst_h, hz_f,
                 idxs_v, idxd_v, gis0, gis1, gid0, gid1,
                 gh0, gh1, gz0, gz1, hzf0, hzf1,
                 sh0, sh1, sz0, sz1, so0, so1):
  c = lax.axis_index("c")
  s = lax.axis_index("s")
  w = c * NS + s
  bw = w * WR

  for j in range(4):
    pltpu.sync_copy(src_h.at[pl.ds(j * RQ + bw, IQS)],
                    idxs_v.at[pl.ds(j * IQS, IQS)])
    pltpu.sync_copy(dst_h.at[pl.ds(j * RQ + bw, IQS)],
                    idxd_v.at[pl.ds(j * IQS, IQS)])

  gis = (gis0, gis1)
  gid = (gid0, gid1)
  gh = (gh0, gh1)
  gz = (gz0, gz1)
  hzf = (hzf0, hzf1)
  hsem = (sh0, sh1)
  zsem = (sz0, sz1)
  osem = (so0, so1)

  def build_gidx(i, b):
    for j in range(4):
      for h2 in range(2):
        off = j * IQS + i * RCH + 16 * h2
        gis[b][pl.ds(32 * j + 16 * h2, 16)] = idxs_v[pl.ds(off, 16)]
        gid[b][pl.ds(32 * j + 16 * h2, 16)] = idxd_v[pl.ds(off, 16)]

  def start_gather(b):
    pltpu.async_copy(feat_h.at[gis[b]], gh[b], hsem[b])
    pltpu.async_copy(feat_h.at[gid[b]], gz[b], zsem[b])

  def wait_gather(b):
    pltpu.make_async_copy(feat_h.at[gis[b]], gh[b], hsem[b]).wait()
    pltpu.make_async_copy(feat_h.at[gid[b]], gz[b], zsem[b]).wait()

  def pack(b):
    for i2 in range(RCH):
      for j in range(4):
        hzf[b][pl.ds(128 * i2 + 32 * j, 16)] = gh[b][32 * j + i2, pl.ds(0, 16)]
        hzf[b][pl.ds(128 * i2 + 32 * j + 16, 16)] = gz[b][32 * j + i2, pl.ds(0, 16)]

  def start_out(i, b):
    pltpu.async_copy(
        hzf[b], hz_f.at[pl.ds((bw + i * RCH) * 128, RCH * 128)], osem[b])

  def wait_out(b):
    pltpu.make_async_copy(
        hzf[b], hz_f.at[pl.ds(bw * 128, RCH * 128)], osem[b]).wait()

  build_gidx(0, 0)
  start_gather(0)
  build_gidx(1, 1)
  start_gather(1)

  def pair(k, _):
    for b in range(2):
      i = 2 * k + b
      wait_gather(b)

      @pl.when(k > 0)
      def _wo():
        wait_out(b)

      pack(b)
      start_out(i, b)
      nxt = i + 2
      if b == 0:
        build_gidx(nxt, b)
        start_gather(b)
      else:
        @pl.when(k < (NCH - 1) // 2 - 1)
        def _ng():
          build_gidx(nxt, b)
          start_gather(b)
    return 0

  lax.fori_loop(0, (NCH - 1) // 2, pair, 0)

  wait_gather(0)
  wait_out(0)
  pack(0)
  start_out(NCH - 1, 0)
  wait_out(1)
  wait_out(0)

  @pl.when(w == NW - 1)
  def _extra():
    for i in (NCH, NCH + 1):
      build_gidx(i, 0)
      start_gather(0)
      wait_gather(0)
      pack(0)
      pltpu.sync_copy(hzf[0], hz_f.at[pl.ds((bw + i * RCH) * 128, RCH * 128)])


def _sc_gather(feat, src, dst):
  return pl.kernel(
      _gather_body,
      out_type=jax.ShapeDtypeStruct((N_EDGES * 32,), jnp.float32),
      mesh=_mesh,
      compiler_params=_sc_params,
      scratch_types=[
          pltpu.VMEM((4 * IQS,), jnp.int32),
          pltpu.VMEM((4 * IQS,), jnp.int32),
          pltpu.VMEM((128,), jnp.int32),
          pltpu.VMEM((128,), jnp.int32),
          pltpu.VMEM((128,), jnp.int32),
          pltpu.VMEM((128,), jnp.int32),
          pltpu.VMEM((128, F), jnp.float32),
          pltpu.VMEM((128, F), jnp.float32),
          pltpu.VMEM((128, F), jnp.float32),
          pltpu.VMEM((128, F), jnp.float32),
          pltpu.VMEM((RCH * 128,), jnp.float32),
          pltpu.VMEM((RCH * 128,), jnp.float32),
      ] + [pltpu.SemaphoreType.DMA] * 6,
  )(feat, src, dst)


RB = 1000
NBLK = RQ // RB


def _k2_body(hz_ref, ef0_ref, ef1_ref, ef2_ref, ef3_ref,
             et0_ref, et1_ref, et2_ref, et3_ref,
             wa_ref, ba_ref, wc0_ref, wc1_ref, bc0_ref, bc1_ref,
             rm_ref, tm_ref, out_ref):
  x = hz_ref[...].reshape(RB, 128)
  a = jnp.dot(x, wa_ref[...], preferred_element_type=jnp.float32) + ba_ref[...]
  a = jnp.where(a >= 0.0, a, NEG_SLOPE * a)
  exa = jnp.exp(a)
  efr = (ef0_ref, ef1_ref, ef2_ref, ef3_ref)
  etr = (et0_ref, et1_ref, et2_ref, et3_ref)
  lane = lax.broadcasted_iota(jnp.int32, (RB, F), 1)
  parts = []
  for j in range(4):
    h = x[:, 32 * j:32 * j + F]
    ef = efr[j][...]
    p = (jnp.dot(ef, rm_ref[...], preferred_element_type=jnp.float32) *
         jnp.dot(h, tm_ref[...], preferred_element_type=jnp.float32))
    v0 = (jnp.dot(p, wc0_ref[...], preferred_element_type=jnp.float32) +
          jnp.dot(h, bc0_ref[...], preferred_element_type=jnp.float32))
    v1 = (jnp.dot(p, wc1_ref[...], preferred_element_type=jnp.float32) +
          jnp.dot(h, bc1_ref[...], preferred_element_type=jnp.float32))
    m0 = (etr[j][...] == 0).astype(jnp.float32)
    ex = exa[:, j:j + 1]
    msg = ex * (m0 * v0 + (1.0 - m0) * v1)
    exl = jnp.where(lane == 0, jnp.broadcast_to(ex, (RB, F)), 0.0)
    parts.append(msg)
    parts.append(exl)
  out = jnp.concatenate(parts, axis=1)
  out_ref[...] = out.reshape(RB * 128)


def _tc_dense(hz_f, efeat, et2, wa, ba, wc0, wc1, bc0, bc1, rm, tm):
  ef_spec = [pl.BlockSpec((RB, F), lambda i, J=j: (J * NBLK + i, 0))
             for j in range(4)]
  et_spec = [pl.BlockSpec((RB, 1), lambda i, J=j: (J * NBLK + i, 0))
             for j in range(4)]
  return pl.pallas_call(
      _k2_body,
      grid=(NBLK,),
      in_specs=[pl.BlockSpec((RB * 128,), lambda i: (i,))] + ef_spec + et_spec + [
          pl.BlockSpec((128, 4), lambda i: (0, 0)),
          pl.BlockSpec((1, 1), lambda i: (0, 0)),
          pl.BlockSpec((F * F, F), lambda i: (0, 0)),
          pl.BlockSpec((F * F, F), lambda i: (0, 0)),
          pl.BlockSpec((F, F), lambda i: (0, 0)),
          pl.BlockSpec((F, F), lambda i: (0, 0)),
          pl.BlockSpec((F, F * F), lambda i: (0, 0)),
          pl.BlockSpec((F, F * F), lambda i: (0, 0)),
      ],
      out_specs=pl.BlockSpec((RB * 128,), lambda i: (i,)),
      out_shape=jax.ShapeDtypeStruct((N_EDGES * 32,), jnp.float32),
  )(hz_f, efeat, efeat, efeat, efeat, et2, et2, et2, et2,
    wa, ba, wc0, wc1, bc0, bc1, rm, tm)


def _scatter_body(pay_f, dst_h, et_h, spart_f,
                  idxd_v, idxt_v, r0, r1, pf0, pf1, sb0, sb1,
                  wb2d, wbf, sacc, ps0, ps1, cs0, cs1):
  c = lax.axis_index("c")
  s = lax.axis_index("s")
  w = c * NS + s
  bw = w * WR

  for i in range(128):
    sb0[i, pl.ds(0, 16)] = jnp.zeros((16,), jnp.float32)
    sb0[i, pl.ds(16, 16)] = jnp.zeros((16,), jnp.float32)
  for m in range(9):
    pltpu.sync_copy(sb0.at[pl.ds(0, 128)],
                    sacc.at[pl.ds(s * ZPT + m * 128, 128)])
  pltpu.sync_copy(sb0.at[pl.ds(0, ZPT - 9 * 128)],
                  sacc.at[pl.ds(s * ZPT + 9 * 128, ZPT - 9 * 128)])

  plsc.subcore_barrier()

  for j in range(4):
    pltpu.sync_copy(dst_h.at[pl.ds(j * RQ + bw, IQS)],
                    idxd_v.at[pl.ds(j * IQS, IQS)])
    pltpu.sync_copy(et_h.at[pl.ds(j * RQ + bw, IQS)],
                    idxt_v.at[pl.ds(j * IQS, IQS)])

  rb = (r0, r1)
  pf = (pf0, pf1)
  sb = (sb0, sb1)
  psem = (ps0, ps1)
  csem = (cs0, cs1)

  def build_ridx(i, b):
    for j in range(4):
      for h2 in range(2):
        off = j * IQS + i * RCH + 16 * h2
        d16 = idxd_v[pl.ds(off, 16)]
        t16 = idxt_v[pl.ds(off, 16)]
        rb[b][pl.ds(32 * j + 16 * h2, 16)] = d16 + t16 * N_NODES

  def start_pay(i, b):
    pltpu.async_copy(
        pay_f.at[pl.ds((bw + i * RCH) * 128, RCH * 128)], pf[b], psem[b])

  def wait_pay(b):
    pltpu.make_async_copy(
        pay_f.at[pl.ds(bw * 128, RCH * 128)], pf[b], psem[b]).wait()

  def unpack(b):
    for i2 in range(RCH):
      for j in range(4):
        sb[b][32 * j + i2, pl.ds(0, 16)] = pf[b][pl.ds(128 * i2 + 32 * j, 16)]
        sb[b][32 * j + i2, pl.ds(16, 16)] = pf[b][pl.ds(128 * i2 + 32 * j + 16, 16)]

  def start_scat(b):
    pltpu.async_copy(sb[b], sacc.at[rb[b]], csem[b], add=True)

  def wait_scat(b):
    pltpu.make_async_copy(sb[b], sacc.at[rb[b]], csem[b]).wait()

  start_pay(0, 0)
  start_pay(1, 1)

  def pair(k, _):
    for b in range(2):
      i = 2 * k + b
      wait_pay(b)

      @pl.when(k > 0)
      def _ws():
        wait_scat(b)

      unpack(b)
      build_ridx(i, b)
      start_scat(b)
      nxt = i + 2
      if b == 0:
        start_pay(nxt, b)
      else:
        @pl.when(k < (NCH - 1) // 2 - 1)
        def _np():
          start_pay(nxt, b)
    return 0

  lax.fori_loop(0, (NCH - 1) // 2, pair, 0)

  wait_pay(0)
  wait_scat(0)
  unpack(0)
  build_ridx(NCH - 1, 0)
  start_scat(0)
  wait_scat(1)
  wait_scat(0)

  @pl.when(w == NW - 1)
  def _extra():
    for i in (NCH, NCH + 1):
      start_pay(i, 0)
      wait_pay(0)
      unpack(0)
      build_ridx(i, 0)
      pltpu.sync_copy(sb[0], sacc.at[rb[0]], add=True)

  plsc.subcore_barrier()

  def wb_chunk(q0, nrows):
    pltpu.sync_copy(sacc.at[pl.ds(q0, nrows)], wb2d.at[pl.ds(0, nrows)])
    for r in range(nrows):
      wbf[pl.ds(32 * r, 16)] = wb2d[r, pl.ds(0, 16)]
      wbf[pl.ds(32 * r + 16, 16)] = wb2d[r, pl.ds(16, 16)]
    pltpu.sync_copy(wbf.at[pl.ds(0, nrows * 32)],
                    spart_f.at[pl.ds((c * ROWS2 + q0) * 32, nrows * 32)])

  @pl.when(s < NS - 1)
  def _wb():
    def wbody(m, _):
      wb_chunk(s * WPT + m * 96, 96)
      return 0
    lax.fori_loop(0, 13, wbody, 0)

  @pl.when(s == NS - 1)
  def _wb_last():
    def wbody(m, _):
      wb_chunk((NS - 1) * WPT + m * 128, 128)
      return 0
    lax.fori_loop(0, 10, wbody, 0)


def _sc_scatter(payload_f, dst, et):
  return pl.kernel(
      _scatter_body,
      out_type=jax.ShapeDtypeStruct((NC * ROWS2 * PAYW,), jnp.float32),
      mesh=_mesh,
      compiler_params=_sc_params,
      scratch_types=[
          pltpu.VMEM((4 * IQS,), jnp.int32),
          pltpu.VMEM((4 * IQS,), jnp.int32),
          pltpu.VMEM((128,), jnp.int32),
          pltpu.VMEM((128,), jnp.int32),
          pltpu.VMEM((RCH * 128,), jnp.float32),
          pltpu.VMEM((RCH * 128,), jnp.float32),
          pltpu.VMEM((128, PAYW), jnp.float32),
          pltpu.VMEM((128, PAYW), jnp.float32),
          pltpu.VMEM((128, PAYW), jnp.float32),
          pltpu.VMEM((128 * PAYW,), jnp.float32),
          pltpu.VMEM_SHARED((SROWS, PAYW), jnp.float32),
      ] + [pltpu.SemaphoreType.DMA] * 4,
  )(payload_f, dst, et)


def _k4_body(sp0_ref, sp1_ref, bias_ref, out_ref):
  x = (sp0_ref[...].reshape(ROWS2 // 4, 128) +
       sp1_ref[...].reshape(ROWS2 // 4, 128))
  x0 = x[0:N_NODES // 4]
  x1 = x[N_NODES // 4:2 * (N_NODES // 4)]
  for u in range(4):
    m0 = x0[:, 32 * u:32 * u + F]
    d0 = x0[:, 32 * u + F:32 * u + F + 1]
    m1 = x1[:, 32 * u:32 * u + F]
    d1 = x1[:, 32 * u + F:32 * u + F + 1]
    out_ref[:, F * u:F * (u + 1)] = (m0 / jnp.where(d0 > 0, d0, 1.0) +
                                     m1 / jnp.where(d1 > 0, d1, 1.0) +
                                     bias_ref[...])


def _tc_finalize(spart_f, bias2):
  half = ROWS2 * PAYW
  return pl.pallas_call(
      _k4_body,
      grid=(1,),
      in_specs=[pl.BlockSpec((half,), lambda i: (0,)),
                pl.BlockSpec((half,), lambda i: (1,)),
                pl.BlockSpec((1, F), lambda i: (0, 0))],
      out_specs=pl.BlockSpec((N_NODES // 4, 4 * F), lambda i: (0, 0)),
      out_shape=jax.ShapeDtypeStruct((N_NODES // 4, 4 * F), jnp.float32),
  )(spart_f, spart_f, bias2)


def kernel(feat, efeat, W_attn, b_attn, W_e1, b_e1, W_e2, b_e2, bias,
           edge_index, etype):
  src = edge_index[0].astype(jnp.int32)
  dst = edge_index[1].astype(jnp.int32)
  et = etype.astype(jnp.int32)
  hz_f = _sc_gather(feat, src, dst)
  wc0 = W_e1.reshape(F * F, F)
  wc1 = W_e2.reshape(F * F, F)
  bc0 = b_e1.reshape(F, F)
  bc1 = b_e2.reshape(F, F)
  wa = jnp.kron(jnp.eye(4, dtype=jnp.float32), W_attn)
  ba = b_attn.reshape(1, 1)
  col = jnp.arange(F * F, dtype=jnp.int32)[None, :]
  row = jnp.arange(F, dtype=jnp.int32)[:, None]
  rm = (col // F == row).astype(jnp.float32)
  tm = (col % F == row).astype(jnp.float32)
  payload_f = _tc_dense(hz_f, efeat, et.reshape(-1, 1), wa, ba,
                        wc0, wc1, bc0, bc1, rm, tm)
  spart_f = _sc_scatter(payload_f, dst, et)
  out64 = _tc_finalize(spart_f, bias.reshape(1, F))
  return out64.reshape(N_NODES, F)


# speedup vs baseline: 1.8283x; 1.0746x over previous
import jax
import jax.numpy as jnp
from jax import lax
from jax.experimental import pallas as pl
from jax.experimental.pallas import tpu as pltpu
from jax.experimental.pallas import tpu_sc as plsc

N_NODES = 10000
N_EDGES = 160000
F = 16
NEG_SLOPE = 0.01

NC, NS = 2, 16
NW = NC * NS
RQ = N_EDGES // 4
RB = 1600
NBLK = RQ // RB
BLK_E = 4 * RB
WR = 1248
WR_LAST = RQ - (NW - 1) * WR
RCH = 32
NCH = WR // RCH
NCH_LAST = WR_LAST // RCH
ROWS2 = 2 * N_NODES
SROWS = ROWS2 + 96
ZPT = SROWS // NS
WPT = 1248
PAYW = 2 * F

_mesh = plsc.VectorSubcoreMesh(
    core_axis_name="c", subcore_axis_name="s", num_cores=NC, num_subcores=NS)
_sc_params = pltpu.CompilerParams(use_tc_tiling_on_sc=False)


def _gather_body(feat_h, src_h, dst_h, hz_f,
                 idxs_v, idxd_v, gis0, gis1, gid0, gid1,
                 gh0, gh1, gz0, gz1, hzf0, hzf1,
                 sh0, sh1, sz0, sz1, so0, so1):
  c = lax.axis_index("c")
  s = lax.axis_index("s")
  w = c * NS + s
  bw = w * WR
  ifirst = bw // RB

  for ib in range(2):
    ibl = jnp.minimum(ifirst + ib, NBLK - 1)
    for j in range(4):
      pltpu.sync_copy(src_h.at[pl.ds(ibl * BLK_E + j * RB, RB)],
                      idxs_v.at[pl.ds((ib * 4 + j) * RB, RB)])
      pltpu.sync_copy(dst_h.at[pl.ds(ibl * BLK_E + j * RB, RB)],
                      idxd_v.at[pl.ds((ib * 4 + j) * RB, RB)])

  gis = (gis0, gis1)
  gid = (gid0, gid1)
  gh = (gh0, gh1)
  gz = (gz0, gz1)
  hzf = (hzf0, hzf1)
  hsem = (sh0, sh1)
  zsem = (sz0, sz1)
  osem = (so0, so1)

  def build_gidx(i, b):
    g0 = bw + i * RCH
    blk = g0 // RB
    base = (blk - ifirst) * 4 * RB + (g0 - blk * RB)
    for j in range(4):
      for h2 in range(2):
        off = base + j * RB + 16 * h2
        gis[b][pl.ds(32 * j + 16 * h2, 16)] = idxs_v[pl.ds(off, 16)]
        gid[b][pl.ds(32 * j + 16 * h2, 16)] = idxd_v[pl.ds(off, 16)]

  def start_gather(b):
    pltpu.async_copy(feat_h.at[gis[b]], gh[b], hsem[b])
    pltpu.async_copy(feat_h.at[gid[b]], gz[b], zsem[b])

  def wait_gather(b):
    pltpu.make_async_copy(feat_h.at[gis[b]], gh[b], hsem[b]).wait()
    pltpu.make_async_copy(feat_h.at[gid[b]], gz[b], zsem[b]).wait()

  def pack(b):
    for i2 in range(RCH):
      for j in range(4):
        hzf[b][pl.ds(128 * i2 + 32 * j, 16)] = gh[b][32 * j + i2, pl.ds(0, 16)]
        hzf[b][pl.ds(128 * i2 + 32 * j + 16, 16)] = gz[b][32 * j + i2, pl.ds(0, 16)]

  def start_out(i, b):
    pltpu.async_copy(
        hzf[b], hz_f.at[pl.ds((bw + i * RCH) * 128, RCH * 128)], osem[b])

  def wait_out(b):
    pltpu.make_async_copy(
        hzf[b], hz_f.at[pl.ds(bw * 128, RCH * 128)], osem[b]).wait()

  build_gidx(0, 0)
  start_gather(0)
  build_gidx(1, 1)
  start_gather(1)

  def pair(k, _):
    for b in range(2):
      i = 2 * k + b
      wait_gather(b)

      @pl.when(k > 0)
      def _wo():
        wait_out(b)

      pack(b)
      start_out(i, b)
      nxt = i + 2
      if b == 0:
        build_gidx(nxt, b)
        start_gather(b)
      else:
        @pl.when(k < (NCH - 1) // 2 - 1)
        def _ng():
          build_gidx(nxt, b)
          start_gather(b)
    return 0

  lax.fori_loop(0, (NCH - 1) // 2, pair, 0)

  wait_gather(0)
  wait_out(0)
  pack(0)
  start_out(NCH - 1, 0)
  wait_out(1)
  wait_out(0)

  @pl.when(w == NW - 1)
  def _extra():
    for i in (NCH, NCH + 1):
      build_gidx(i, 0)
      start_gather(0)
      wait_gather(0)
      pack(0)
      pltpu.sync_copy(hzf[0], hz_f.at[pl.ds((bw + i * RCH) * 128, RCH * 128)])


def _sc_gather(feat, src, dst):
  return pl.kernel(
      _gather_body,
      out_type=jax.ShapeDtypeStruct((N_EDGES * 32,), jnp.float32),
      mesh=_mesh,
      compiler_params=_sc_params,
      scratch_types=[
          pltpu.VMEM((8 * RB,), jnp.int32),
          pltpu.VMEM((8 * RB,), jnp.int32),
          pltpu.VMEM((128,), jnp.int32),
          pltpu.VMEM((128,), jnp.int32),
          pltpu.VMEM((128,), jnp.int32),
          pltpu.VMEM((128,), jnp.int32),
          pltpu.VMEM((128, F), jnp.float32),
          pltpu.VMEM((128, F), jnp.float32),
          pltpu.VMEM((128, F), jnp.float32),
          pltpu.VMEM((128, F), jnp.float32),
          pltpu.VMEM((RCH * 128,), jnp.float32),
          pltpu.VMEM((RCH * 128,), jnp.float32),
      ] + [pltpu.SemaphoreType.DMA] * 6,
  )(feat, src, dst)


def _k2_body(hz_ref, ef_ref, et_ref,
             wa_ref, ba_ref, wc0_ref, wc1_ref, bc0_ref, bc1_ref,
             rm_ref, tm_ref, out_ref):
  x = hz_ref[...].reshape(RB, 128)
  a = jnp.dot(x, wa_ref[...], preferred_element_type=jnp.float32) + ba_ref[...]
  a = jnp.where(a >= 0.0, a, NEG_SLOPE * a)
  exa = jnp.exp(a)
  efa = ef_ref[...]
  eta = et_ref[...]
  lane = lax.broadcasted_iota(jnp.int32, (RB, F), 1)
  parts = []
  for j in range(4):
    h = x[:, 32 * j:32 * j + F]
    ef = efa[j * RB:(j + 1) * RB]
    p = (jnp.dot(ef, rm_ref[...], preferred_element_type=jnp.float32) *
         jnp.dot(h, tm_ref[...], preferred_element_type=jnp.float32))
    v0 = (jnp.dot(p, wc0_ref[...], preferred_element_type=jnp.float32) +
          jnp.dot(h, bc0_ref[...], preferred_element_type=jnp.float32))
    v1 = (jnp.dot(p, wc1_ref[...], preferred_element_type=jnp.float32) +
          jnp.dot(h, bc1_ref[...], preferred_element_type=jnp.float32))
    m0 = (eta[j * RB:(j + 1) * RB] == 0).astype(jnp.float32)
    ex = exa[:, j:j + 1]
    msg = ex * (m0 * v0 + (1.0 - m0) * v1)
    exl = jnp.where(lane == 0, jnp.broadcast_to(ex, (RB, F)), 0.0)
    parts.append(msg)
    parts.append(exl)
  out = jnp.concatenate(parts, axis=1)
  out_ref[...] = out.reshape(RB * 128)


def _tc_dense(hz_f, efeat, et8, wa, ba, wc0, wc1, bc0, bc1, rm, tm):
  return pl.pallas_call(
      _k2_body,
      grid=(NBLK,),
      in_specs=[
          pl.BlockSpec((RB * 128,), lambda i: (i,)),
          pl.BlockSpec((BLK_E, F), lambda i: (i, 0)),
          pl.BlockSpec((BLK_E, 1), lambda i: (i, 0)),
          pl.BlockSpec((128, 4), lambda i: (0, 0)),
          pl.BlockSpec((1, 1), lambda i: (0, 0)),
          pl.BlockSpec((F * F, F), lambda i: (0, 0)),
          pl.BlockSpec((F * F, F), lambda i: (0, 0)),
          pl.BlockSpec((F, F), lambda i: (0, 0)),
          pl.BlockSpec((F, F), lambda i: (0, 0)),
          pl.BlockSpec((F, F * F), lambda i: (0, 0)),
          pl.BlockSpec((F, F * F), lambda i: (0, 0)),
      ],
      out_specs=pl.BlockSpec((RB * 128,), lambda i: (i,)),
      out_shape=jax.ShapeDtypeStruct((N_EDGES * 32,), jnp.float32),
  )(hz_f, efeat, et8, wa, ba, wc0, wc1, bc0, bc1, rm, tm)


def _scatter_body(pay_f, dst_h, et_h, spart_f,
                  idxd_v, idxt_v, r0, r1, pf0, pf1, sb0, sb1,
                  wb2d, wbf, sacc, ps0, ps1, cs0, cs1):
  c = lax.axis_index("c")
  s = lax.axis_index("s")
  w = c * NS + s
  bw = w * WR
  ifirst = bw // RB

  for i in range(128):
    sb0[i, pl.ds(0, 16)] = jnp.zeros((16,), jnp.float32)
    sb0[i, pl.ds(16, 16)] = jnp.zeros((16,), jnp.float32)
  for m in range(9):
    pltpu.sync_copy(sb0.at[pl.ds(0, 128)],
                    sacc.at[pl.ds(s * ZPT + m * 128, 128)])
  pltpu.sync_copy(sb0.at[pl.ds(0, ZPT - 9 * 128)],
                  sacc.at[pl.ds(s * ZPT + 9 * 128, ZPT - 9 * 128)])

  plsc.subcore_barrier()

  for ib in range(2):
    ibl = jnp.minimum(ifirst + ib, NBLK - 1)
    for j in range(4):
      pltpu.sync_copy(dst_h.at[pl.ds(ibl * BLK_E + j * RB, RB)],
                      idxd_v.at[pl.ds((ib * 4 + j) * RB, RB)])
      pltpu.sync_copy(et_h.at[pl.ds(ibl * BLK_E + j * RB, RB)],
                      idxt_v.at[pl.ds((ib * 4 + j) * RB, RB)])

  rb = (r0, r1)
  pf = (pf0, pf1)
  sb = (sb0, sb1)
  psem = (ps0, ps1)
  csem = (cs0, cs1)

  def build_ridx(i, b):
    g0 = bw + i * RCH
    blk = g0 // RB
    base = (blk - ifirst) * 4 * RB + (g0 - blk * RB)
    for j in range(4):
      for h2 in range(2):
        off = base + j * RB + 16 * h2
        d16 = idxd_v[pl.ds(off, 16)]
        t16 = idxt_v[pl.ds(off, 16)]
        rb[b][pl.ds(32 * j + 16 * h2, 16)] = d16 + t16 * N_NODES

  def start_pay(i, b):
    pltpu.async_copy(
        pay_f.at[pl.ds((bw + i * RCH) * 128, RCH * 128)], pf[b], psem[b])

  def wait_pay(b):
    pltpu.make_async_copy(
        pay_f.at[pl.ds(bw * 128, RCH * 128)], pf[b], psem[b]).wait()

  def unpack(b):
    for i2 in range(RCH):
      for j in range(4):
        sb[b][32 * j + i2, pl.ds(0, 16)] = pf[b][pl.ds(128 * i2 + 32 * j, 16)]
        sb[b][32 * j + i2, pl.ds(16, 16)] = pf[b][pl.ds(128 * i2 + 32 * j + 16, 16)]

  def start_scat(b):
    pltpu.async_copy(sb[b], sacc.at[rb[b]], csem[b], add=True)

  def wait_scat(b):
    pltpu.make_async_copy(sb[b], sacc.at[rb[b]], csem[b]).wait()

  start_pay(0, 0)
  start_pay(1, 1)

  def pair(k, _):
    for b in range(2):
      i = 2 * k + b
      wait_pay(b)

      @pl.when(k > 0)
      def _ws():
        wait_scat(b)

      unpack(b)
      build_ridx(i, b)
      start_scat(b)
      nxt = i + 2
      if b == 0:
        start_pay(nxt, b)
      else:
        @pl.when(k < (NCH - 1) // 2 - 1)
        def _np():
          start_pay(nxt, b)
    return 0

  lax.fori_loop(0, (NCH - 1) // 2, pair, 0)

  wait_pay(0)
  wait_scat(0)
  unpack(0)
  build_ridx(NCH - 1, 0)
  start_scat(0)
  wait_scat(1)
  wait_scat(0)

  @pl.when(w == NW - 1)
  def _extra():
    for i in (NCH, NCH + 1):
      start_pay(i, 0)
      wait_pay(0)
      unpack(0)
      build_ridx(i, 0)
      pltpu.sync_copy(sb[0], sacc.at[rb[0]], add=True)

  plsc.subcore_barrier()

  def wb_chunk(q0, nrows):
    pltpu.sync_copy(sacc.at[pl.ds(q0, nrows)], wb2d.at[pl.ds(0, nrows)])
    for r in range(nrows):
      wbf[pl.ds(32 * r, 16)] = wb2d[r, pl.ds(0, 16)]
      wbf[pl.ds(32 * r + 16, 16)] = wb2d[r, pl.ds(16, 16)]
    pltpu.sync_copy(wbf.at[pl.ds(0, nrows * 32)],
                    spart_f.at[pl.ds((c * ROWS2 + q0) * 32, nrows * 32)])

  @pl.when(s < NS - 1)
  def _wb():
    def wbody(m, _):
      wb_chunk(s * WPT + m * 96, 96)
      return 0
    lax.fori_loop(0, 13, wbody, 0)

  @pl.when(s == NS - 1)
  def _wb_last():
    def wbody(m, _):
      wb_chunk((NS - 1) * WPT + m * 128, 128)
      return 0
    lax.fori_loop(0, 10, wbody, 0)


def _sc_scatter(payload_f, dst, et):
  return pl.kernel(
      _scatter_body,
      out_type=jax.ShapeDtypeStruct((NC * ROWS2 * PAYW,), jnp.float32),
      mesh=_mesh,
      compiler_params=_sc_params,
      scratch_types=[
          pltpu.VMEM((8 * RB,), jnp.int32),
          pltpu.VMEM((8 * RB,), jnp.int32),
          pltpu.VMEM((128,), jnp.int32),
          pltpu.VMEM((128,), jnp.int32),
          pltpu.VMEM((RCH * 128,), jnp.float32),
          pltpu.VMEM((RCH * 128,), jnp.float32),
          pltpu.VMEM((128, PAYW), jnp.float32),
          pltpu.VMEM((128, PAYW), jnp.float32),
          pltpu.VMEM((128, PAYW), jnp.float32),
          pltpu.VMEM((128 * PAYW,), jnp.float32),
          pltpu.VMEM_SHARED((SROWS, PAYW), jnp.float32),
      ] + [pltpu.SemaphoreType.DMA] * 4,
  )(payload_f, dst, et)


def _k4_body(sp0_ref, sp1_ref, bias_ref, out_ref):
  x = (sp0_ref[...].reshape(ROWS2 // 4, 128) +
       sp1_ref[...].reshape(ROWS2 // 4, 128))
  x0 = x[0:N_NODES // 4]
  x1 = x[N_NODES // 4:2 * (N_NODES // 4)]
  for u in range(4):
    m0 = x0[:, 32 * u:32 * u + F]
    d0 = x0[:, 32 * u + F:32 * u + F + 1]
    m1 = x1[:, 32 * u:32 * u + F]
    d1 = x1[:, 32 * u + F:32 * u + F + 1]
    out_ref[:, F * u:F * (u + 1)] = (m0 / jnp.where(d0 > 0, d0, 1.0) +
                                     m1 / jnp.where(d1 > 0, d1, 1.0) +
                                     bias_ref[...])


def _tc_finalize(spart_f, bias2):
  half = ROWS2 * PAYW
  return pl.pallas_call(
      _k4_body,
      grid=(1,),
      in_specs=[pl.BlockSpec((half,), lambda i: (0,)),
                pl.BlockSpec((half,), lambda i: (1,)),
                pl.BlockSpec((1, F), lambda i: (0, 0))],
      out_specs=pl.BlockSpec((N_NODES // 4, 4 * F), lambda i: (0, 0)),
      out_shape=jax.ShapeDtypeStruct((N_NODES // 4, 4 * F), jnp.float32),
  )(spart_f, spart_f, bias2)


def kernel(feat, efeat, W_attn, b_attn, W_e1, b_e1, W_e2, b_e2, bias,
           edge_index, etype):
  src = edge_index[0].astype(jnp.int32)
  dst = edge_index[1].astype(jnp.int32)
  et = etype.astype(jnp.int32)
  hz_f = _sc_gather(feat, src, dst)
  wc0 = W_e1.reshape(F * F, F)
  wc1 = W_e2.reshape(F * F, F)
  bc0 = b_e1.reshape(F, F)
  bc1 = b_e2.reshape(F, F)
  wa = jnp.kron(jnp.eye(4, dtype=jnp.float32), W_attn)
  ba = b_attn.reshape(1, 1)
  col = jnp.arange(F * F, dtype=jnp.int32)[None, :]
  row = jnp.arange(F, dtype=jnp.int32)[:, None]
  rm = (col // F == row).astype(jnp.float32)
  tm = (col % F == row).astype(jnp.float32)
  et8 = et.reshape(-1, 1)
  payload_f = _tc_dense(hz_f, efeat, et8, wa, ba,
                        wc0, wc1, bc0, bc1, rm, tm)
  spart_f = _sc_scatter(payload_f, dst, et)
  out64 = _tc_finalize(spart_f, bias.reshape(1, F))
  return out64.reshape(N_NODES, F)
